# baseline (speedup 1.0000x reference)
import jax
import jax.numpy as jnp
from jax.experimental import pallas as pl
from jax.experimental.pallas import tpu as pltpu
from jax.experimental.pallas import tpu_sc as plsc

_B = 16384 * 200
_D = 32
_W = 512

_mesh = plsc.VectorSubcoreMesh(core_axis_name="core", subcore_axis_name="subcore")


@jax.jit
def _gather(table, idx):
  @pl.kernel(
      out_type=jax.ShapeDtypeStruct((_B, _D), jnp.float32),
      mesh=_mesh,
      compiler_params=pltpu.CompilerParams(use_tc_tiling_on_sc=False),
  )
  def k(table_hbm, i_hbm, o_hbm):
    def body(i_vmem, o_vmem):
      pltpu.sync_copy(table_hbm.at[i_vmem.at[0]], o_vmem)

    pltpu.emit_pipeline(
        body,
        grid=(_B // _W,),
        in_specs=[pl.BlockSpec((1, _W), index_map=lambda i: (0, i))],
        out_specs=[pl.BlockSpec((_W, _D), index_map=lambda i: (i, 0))],
        core_axis_name=("core", "subcore"),
        dimension_semantics=(pltpu.PARALLEL,),
    )(i_hbm, o_hbm)

  return k(table, idx)


def kernel(src, table):
  idx = src.reshape(1, _B)
  out = _gather(table, idx)
  return out.reshape(src.shape[0], src.shape[1], _D)


# speedup vs baseline: 1.0896x; 1.0896x over previous
import jax
import jax.numpy as jnp
from jax.experimental import pallas as pl
from jax.experimental.pallas import tpu as pltpu
from jax.experimental.pallas import tpu_sc as plsc

_B = 16384 * 200
_D = 32
_W = 128

_mesh = plsc.VectorSubcoreMesh(core_axis_name="core", subcore_axis_name="subcore")


@jax.jit
def _gather(table, idx):
  @pl.kernel(
      out_type=jax.ShapeDtypeStruct((_B, _D), jnp.float32),
      mesh=_mesh,
      compiler_params=pltpu.CompilerParams(use_tc_tiling_on_sc=False),
  )
  def k(table_hbm, i_hbm, o_hbm):
    def body(i_vmem, o_vmem):
      pltpu.sync_copy(table_hbm.at[i_vmem.at[0]], o_vmem)

    pltpu.emit_pipeline(
        body,
        grid=(_B // _W,),
        in_specs=[pl.BlockSpec((1, _W), index_map=lambda i: (0, i))],
        out_specs=[pl.BlockSpec((_W, _D), index_map=lambda i: (i, 0))],
        core_axis_name=("core", "subcore"),
        dimension_semantics=(pltpu.PARALLEL,),
    )(i_hbm, o_hbm)

  return k(table, idx)


def kernel(src, table):
  idx = src.reshape(1, _B)
  out = _gather(table, idx)
  return out.reshape(src.shape[0], src.shape[1], _D)
